# Initial kernel scaffold; baseline (speedup 1.0000x reference)
import jax
import jax.numpy as jnp
from jax.experimental import pallas as pl


def kernel(hidden_preactivation_BH):
    raise NotImplementedError("write your pallas kernel here")


# speedup vs baseline: 16.9745x; 16.9745x over previous
import functools

import jax
import jax.numpy as jnp
from jax import lax
from jax.experimental import pallas as pl
from jax.experimental.pallas import tpu as pltpu
from jax.experimental.pallas import tpu_sc as plsc

_K_PER_EXAMPLE = 32
_NC, _NS = 2, 16
_NW = _NC * _NS
_LANES = 16
_CH = 16384

_B1 = 1 << 12
_B2 = 1 << 12
_B3 = 1 << 8
_CAND_CAP = 32


def _skey(bits):
    return bits ^ lax.shift_right_logical(lax.shift_right_arithmetic(bits, 31), 1)


def _make_hist_kernel(n, pass_id):
    per_w = n // _NW
    nchunk = per_w // _CH
    nbins = (_B1, _B2, _B3)[pass_id - 1]
    hwords = nbins * _LANES
    mesh = plsc.VectorSubcoreMesh(core_axis_name="c", subcore_axis_name="s")
    scratch = [
        pltpu.VMEM((_CH,), jnp.float32),
        pltpu.VMEM((hwords,), jnp.int32),
    ]
    if pass_id > 1:
        scratch.append(pltpu.VMEM((128,), jnp.int32))
    out_type = jax.ShapeDtypeStruct((_NW, hwords), jnp.int32)
    if pass_id == 3:
        out_type = [out_type,
                    jax.ShapeDtypeStruct((_NW, 2 * _CAND_CAP), jnp.int32)]
        scratch.append(pltpu.VMEM((2 * _CAND_CAP,), jnp.int32))

    def body(*refs):
        if pass_id == 1:
            x_hbm, out_hbm, buf, hist = refs
            mbuf = cand = cand_hbm = None
        elif pass_id == 2:
            x_hbm, meta_hbm, out_hbm, buf, hist, mbuf = refs
            cand = cand_hbm = None
        else:
            x_hbm, meta_hbm, out_hbm, cand_hbm, buf, hist, mbuf, cand = refs
        wid = lax.axis_index("s") * _NC + lax.axis_index("c")
        base = wid * per_w
        lane = lax.iota(jnp.int32, _LANES)
        ones = jnp.ones((_LANES,), jnp.int32)
        zeros = jnp.zeros((_LANES,), jnp.int32)
        if pass_id > 1:
            pltpu.sync_copy(meta_hbm.at[0], mbuf)

        def zbody(i, _):
            hist[pl.ds(i * _LANES, _LANES)] = zeros
            return 0

        lax.fori_loop(0, nbins, zbody, 0)
        if pass_id == 3:
            for j in range(2 * _CAND_CAP // _LANES):
                cand[pl.ds(j * _LANES, _LANES)] = zeros
        ref_vec = None if pass_id == 1 else mbuf[pl.ds(0, _LANES)]

        def cbody(c, o):
            pltpu.sync_copy(x_hbm.at[pl.ds(base + c * _CH, _CH)], buf)

            def ibody(i, o):
                v = buf[pl.ds(i * _LANES, _LANES)]
                bits = plsc.bitcast(v, jnp.int32)
                sk = _skey(bits)
                if pass_id == 1:
                    b = lax.shift_right_arithmetic(sk, 20) + 2048
                    plsc.addupdate_scatter(hist, [b * _LANES + lane], ones)
                elif pass_id == 2:
                    match = (lax.shift_right_arithmetic(sk, 20) + 2048) == ref_vec
                    b = lax.shift_right_arithmetic(sk, 8) & 0xFFF
                    plsc.addupdate_scatter(
                        hist, [b * _LANES + lane], ones, mask=match)
                else:
                    match = lax.shift_right_arithmetic(sk, 8) == ref_vec
                    b = sk & 0xFF
                    plsc.addupdate_scatter(
                        hist, [b * _LANES + lane], ones, mask=match)
                    packed = (b << 24) | (base + c * _CH + i * _LANES + lane)
                    plsc.store_compressed(
                        cand.at[pl.ds(o, _LANES)], packed, mask=match)
                    o = jnp.minimum(
                        o + jnp.sum(match.astype(jnp.int32)), _CAND_CAP)
                return o

            return lax.fori_loop(0, _CH // _LANES, ibody, o)

        o = lax.fori_loop(0, nchunk, cbody, jnp.int32(0))
        if pass_id == 3:
            cand[pl.ds(_CAND_CAP, _LANES)] = ones * o
            pltpu.sync_copy(cand, cand_hbm.at[wid])
        pltpu.sync_copy(hist, out_hbm.at[wid])

    return pl.kernel(
        body,
        out_type=out_type,
        mesh=mesh,
        scratch_types=scratch,
        compiler_params=pltpu.CompilerParams(needs_layout_passes=False),
    )


def _suffix_stats(hm, kq):
    r = hm.shape[0]
    c_i = lax.broadcasted_iota(jnp.int32, (128, 8), 0)
    g_i = lax.broadcasted_iota(jnp.int32, (128, 8), 1)
    q_ge = ((c_i >> 4) >= g_i).astype(jnp.float32)
    q_eq = ((c_i >> 4) == g_i).astype(jnp.float32)
    w = lax.dot(hm, q_ge, precision=lax.Precision.HIGHEST)
    cnts = lax.dot(hm, q_eq, precision=lax.Precision.HIGHEST)
    i_i = lax.broadcasted_iota(jnp.int32, (r, r), 0)
    j_i = lax.broadcasted_iota(jnp.int32, (r, r), 1)
    m_gt = (j_i > i_i).astype(jnp.float32)
    texcl = lax.dot(m_gt, w[:, 0:1], precision=lax.Precision.HIGHEST)
    s = w + texcl
    flat = (lax.broadcasted_iota(jnp.int32, (r, 8), 0) * 8
            + lax.broadcasted_iota(jnp.int32, (r, 8), 1)).astype(jnp.float32)
    bsel = jnp.sum((s >= kq).astype(jnp.float32)) - 1.0
    sb = jnp.sum(jnp.where(flat == bsel, s, 0.0))
    cb = jnp.sum(jnp.where(flat == bsel, cnts, 0.0))
    kres = kq - (sb - cb)
    return bsel, kres, sb


def _meta_out(o_ref, v16, v1):
    col = lax.broadcasted_iota(jnp.int32, (8, 128), 1)
    o_ref[...] = jnp.where(col < 16, v16, jnp.where(col == 16, v1, 0))


def _merge1_body(kk, h_ref, o_ref):
    h = jnp.sum(h_ref[...].astype(jnp.float32), axis=0)
    b0, k0, _ = _suffix_stats(h.reshape(_B1 * _LANES // 128, 128),
                              jnp.float32(kk))
    _meta_out(o_ref, b0.astype(jnp.int32), k0.astype(jnp.int32))


def _merge2_body(h_ref, m_ref, o_ref):
    b0 = m_ref[0, 0]
    k0 = m_ref[0, 16].astype(jnp.float32)
    h = jnp.sum(h_ref[...].astype(jnp.float32), axis=0)
    b1, k1, _ = _suffix_stats(h.reshape(_B2 * _LANES // 128, 128), k0)
    p24 = ((b0 - 2048) << 12) | b1.astype(jnp.int32)
    _meta_out(o_ref, p24, k1.astype(jnp.int32))


def _merge3_body(h_ref, c_ref, m_ref, o_ref):
    p24 = m_ref[0, 0]
    k1 = m_ref[0, 16].astype(jnp.float32)
    h = jnp.sum(h_ref[...].astype(jnp.float32), axis=0)
    b2f, _, sb3 = _suffix_stats(h.reshape(_B3 * _LANES // 128, 128), k1)
    b2 = b2f.astype(jnp.int32)
    skey_t = (p24 << 8) | b2
    cands = c_ref[...]
    pk = cands[:, 0:_CAND_CAP]
    cntc = cands[:, _CAND_CAP:_CAND_CAP + 1]
    col = lax.broadcasted_iota(jnp.int32, (_NW, _CAND_CAP), 1)
    valid = col < cntc
    is_tie = valid & (lax.shift_right_logical(pk, 24) == b2)
    idxi = pk & 0xFFFFFF
    mult = jnp.sum(is_tie.astype(jnp.float32))
    need = k1 - sb3 + mult
    prefix = jnp.int32(0)
    for bbit in range(23, -1, -1):
        v_try = prefix | ((1 << bbit) - 1)
        c = jnp.sum(jnp.where(is_tie & (idxi <= v_try), 1.0, 0.0))
        prefix = jnp.where(c >= need, prefix, prefix | (1 << bbit))
    _meta_out(o_ref, skey_t, prefix)


def _mask_body(x_ref, m_ref, o_ref):
    t = m_ref[0, 0]
    i_t = m_ref[0, 16]
    x = x_ref[...]
    sk = _skey(lax.bitcast_convert_type(x, jnp.int32))
    r_i = lax.broadcasted_iota(jnp.int32, x.shape, 0)
    c_i = lax.broadcasted_iota(jnp.int32, x.shape, 1)
    idx = (pl.program_id(0) * x.shape[0] + r_i) * x.shape[1] + c_i
    keep = (sk > t) | ((sk == t) & (idx <= i_t))
    o_ref[...] = jnp.where(keep, x, 0.0)


def kernel(hidden_preactivation_BH):
    x = hidden_preactivation_BH
    bsz, hdim = x.shape
    n = bsz * hdim
    kk = _K_PER_EXAMPLE * bsz
    xf = x.reshape(n)

    h1 = _make_hist_kernel(n, 1)(xf)
    meta1 = pl.pallas_call(
        functools.partial(_merge1_body, kk),
        out_shape=jax.ShapeDtypeStruct((8, 128), jnp.int32),
    )(h1)
    h2 = _make_hist_kernel(n, 2)(xf, meta1)
    meta2 = pl.pallas_call(
        _merge2_body,
        out_shape=jax.ShapeDtypeStruct((8, 128), jnp.int32),
    )(h2, meta1)
    h3, cands = _make_hist_kernel(n, 3)(xf, meta2)
    meta3 = pl.pallas_call(
        _merge3_body,
        out_shape=jax.ShapeDtypeStruct((8, 128), jnp.int32),
    )(h3, cands, meta2)

    rows_per_blk = 128
    grid = bsz // rows_per_blk
    out = pl.pallas_call(
        _mask_body,
        grid=(grid,),
        in_specs=[
            pl.BlockSpec((rows_per_blk, hdim), lambda i: (i, 0)),
            pl.BlockSpec((8, 128), lambda i: (0, 0)),
        ],
        out_specs=pl.BlockSpec((rows_per_blk, hdim), lambda i: (i, 0)),
        out_shape=jax.ShapeDtypeStruct((bsz, hdim), jnp.float32),
    )(x, meta3)
    return out
